# baseline (device time: 439718 ns/iter reference)
import jax
import jax.numpy as jnp
from jax import lax
from jax.experimental import pallas as pl
from jax.experimental.pallas import tpu as pltpu

ROWS = 4096
COLS = 1024
CHUNK = 256
N_CHUNKS = ROWS // CHUNK
W = 768


def kernel(x, dest):
    my_x = lax.axis_index("x")

    keep = (dest == my_x).astype(jnp.int32)
    n_keep = jnp.sum(keep)
    ns = ROWS - n_keep
    order = jnp.argsort(keep, stable=True).astype(jnp.int32)
    ar = jnp.arange(ROWS, dtype=jnp.int32)

    rb = my_x * n_keep
    kb = my_x * ns
    send_ord = order[jnp.clip(ar - rb, 0, ns - 1)]
    keep_ord = order[ns + jnp.clip(ar - kb, 0, n_keep - 1)]
    lo = rb // CHUNK
    hi = (rb + ns + CHUNK - 1) // CHUNK
    klo = kb // CHUNK
    khi = (kb + n_keep + CHUNK - 1) // CHUNK
    recv_base = (1 - my_x) * n_keep
    rlo = recv_base // CHUNK
    rhi = (recv_base + ns + CHUNK - 1) // CHUNK

    def starts_of(ord_arr):
        mins = ord_arr.reshape(N_CHUNKS, CHUNK).min(axis=1)
        return jnp.minimum((mins // 8) * 8, ROWS - W)

    scal = jnp.concatenate(
        [
            jnp.stack([ns, lo, hi, klo, khi, rlo, rhi]),
            starts_of(send_ord),
            starts_of(keep_ord),
        ]
    ).astype(jnp.int32)
    xb = x.astype(jnp.bfloat16)

    def body(scal_ref, sord_ref, kord_ref, xb_ref, out_ref,
             buf_ref, kbuf_ref, recv_ref, send_sems, recv_sems):
        mx = lax.axis_index("x")
        my = lax.axis_index("y")
        mz = lax.axis_index("z")
        peer = (1 - mx, my, mz)

        ns = scal_ref[0]
        nk = ROWS - ns
        lo, hi = scal_ref[1], scal_ref[2]
        klo, khi = scal_ref[3], scal_ref[4]
        rlo, rhi = scal_ref[5], scal_ref[6]

        barrier_sem = pltpu.get_barrier_semaphore()
        pl.semaphore_signal(
            barrier_sem, inc=1, device_id=peer,
            device_id_type=pl.DeviceIdType.MESH,
        )
        pl.semaphore_wait(barrier_sem, 1)

        def chunk_rdma(i):
            return pltpu.make_async_remote_copy(
                src_ref=buf_ref.at[pl.ds(i * CHUNK, CHUNK), :],
                dst_ref=recv_ref.at[pl.ds(i * CHUNK, CHUNK), :],
                send_sem=send_sems.at[i],
                recv_sem=recv_sems.at[i],
                device_id=peer,
                device_id_type=pl.DeviceIdType.MESH,
            )

        def compact_chunk(c, ord_ref, start, dst_ref):
            ords = ord_ref[pl.ds(c * CHUNK, CHUNK), :]
            col = lax.broadcasted_iota(jnp.int32, (CHUNK, W), 1) + start
            p = (ords == col).astype(jnp.bfloat16)
            rows = lax.dot_general(
                p, xb_ref[pl.ds(start, W), :], (((1,), (0,)), ((), ())),
                preferred_element_type=jnp.float32,
            )
            dst_ref[pl.ds(c * CHUNK, CHUNK), :] = rows.astype(jnp.bfloat16)

        for c in range(N_CHUNKS):
            @pl.when((c >= lo) & (c < hi))
            def _(c=c):
                start = pl.multiple_of(scal_ref[7 + c], 8)
                compact_chunk(c, sord_ref, start, buf_ref)
                chunk_rdma(c).start()

        for c in range(N_CHUNKS):
            @pl.when((c >= klo) & (c < khi))
            def _(c=c):
                start = pl.multiple_of(scal_ref[7 + N_CHUNKS + c], 8)
                compact_chunk(c, kord_ref, start, kbuf_ref)

        for c in range(N_CHUNKS):
            @pl.when((c >= rlo) & (c < rhi))
            def _(c=c):
                chunk_rdma(c).wait_recv()

        for c in range(N_CHUNKS):
            @pl.when((c >= lo) & (c < hi))
            def _(c=c):
                chunk_rdma(c).wait_send()

        keep_base = mx * ns
        row = lax.broadcasted_iota(jnp.int32, (ROWS, 1), 0)
        in_keep = (row >= keep_base) & (row < keep_base + nk)
        SLAB = 512
        for s in range(COLS // SLAB):
            cols = pl.ds(s * SLAB, SLAB)
            out_ref[:, cols] = jnp.where(
                in_keep, kbuf_ref[:, cols], recv_ref[:, cols]
            )

    return pl.pallas_call(
        body,
        out_shape=jax.ShapeDtypeStruct((ROWS, COLS), jnp.bfloat16),
        in_specs=[
            pl.BlockSpec(memory_space=pltpu.SMEM),
            pl.BlockSpec(memory_space=pltpu.VMEM),
            pl.BlockSpec(memory_space=pltpu.VMEM),
            pl.BlockSpec(memory_space=pltpu.VMEM),
        ],
        out_specs=pl.BlockSpec(memory_space=pltpu.VMEM),
        scratch_shapes=[
            pltpu.VMEM((ROWS, COLS), jnp.bfloat16),
            pltpu.VMEM((ROWS, COLS), jnp.bfloat16),
            pltpu.VMEM((ROWS, COLS), jnp.bfloat16),
            pltpu.SemaphoreType.DMA((N_CHUNKS,)),
            pltpu.SemaphoreType.DMA((N_CHUNKS,)),
        ],
        compiler_params=pltpu.CompilerParams(
            collective_id=0, vmem_limit_bytes=100 * 1024 * 1024
        ),
    )(scal, send_ord.reshape(ROWS, 1), keep_ord.reshape(ROWS, 1), xb)


# device time: 92817 ns/iter; 4.7375x vs baseline; 4.7375x over previous
import jax
import jax.numpy as jnp
from jax import lax
from jax.experimental import pallas as pl
from jax.experimental.pallas import tpu as pltpu

ROWS = 4096
COLS = 1024
CHUNK = 256
N_CHUNKS = ROWS // CHUNK
W = 768


def kernel(x, dest):
    my_x = lax.axis_index("x")

    keep = (dest == my_x).astype(jnp.int32)
    n_keep = jnp.sum(keep)
    ns = ROWS - n_keep
    order = jnp.argsort(keep, stable=True).astype(jnp.int32)
    ar = jnp.arange(ROWS, dtype=jnp.int32)

    rb = my_x * n_keep
    kb = my_x * ns
    spad = jnp.where(ar < ns, order, order[ns - 1])
    send_ord = jnp.where(ar < rb, order[0], jnp.roll(spad, rb))
    kpart = jnp.roll(order, -ns)
    kpad = jnp.where(ar < n_keep, kpart, order[ROWS - 1])
    keep_ord = jnp.where(ar < kb, order[ns], jnp.roll(kpad, kb))
    lo = rb // CHUNK
    hi = (rb + ns + CHUNK - 1) // CHUNK
    klo = kb // CHUNK
    khi = (kb + n_keep + CHUNK - 1) // CHUNK
    recv_base = (1 - my_x) * n_keep
    rlo = recv_base // CHUNK
    rhi = (recv_base + ns + CHUNK - 1) // CHUNK

    def starts_of(ord_arr):
        mins = ord_arr.reshape(N_CHUNKS, CHUNK).min(axis=1)
        return jnp.minimum((mins // 8) * 8, ROWS - W)

    scal = jnp.concatenate(
        [
            jnp.stack([ns, lo, hi, klo, khi, rlo, rhi]),
            starts_of(send_ord),
            starts_of(keep_ord),
        ]
    ).astype(jnp.int32)
    xb = x.astype(jnp.bfloat16)

    def body(scal_ref, sord_ref, kord_ref, xb_ref, out_ref,
             buf_ref, kbuf_ref, recv_ref, send_sems, recv_sems):
        mx = lax.axis_index("x")
        my = lax.axis_index("y")
        mz = lax.axis_index("z")
        peer = (1 - mx, my, mz)

        ns = scal_ref[0]
        nk = ROWS - ns
        lo, hi = scal_ref[1], scal_ref[2]
        klo, khi = scal_ref[3], scal_ref[4]
        rlo, rhi = scal_ref[5], scal_ref[6]

        barrier_sem = pltpu.get_barrier_semaphore()
        pl.semaphore_signal(
            barrier_sem, inc=1, device_id=peer,
            device_id_type=pl.DeviceIdType.MESH,
        )
        pl.semaphore_wait(barrier_sem, 1)

        def chunk_rdma(i):
            return pltpu.make_async_remote_copy(
                src_ref=buf_ref.at[pl.ds(i * CHUNK, CHUNK), :],
                dst_ref=recv_ref.at[pl.ds(i * CHUNK, CHUNK), :],
                send_sem=send_sems.at[i],
                recv_sem=recv_sems.at[i],
                device_id=peer,
                device_id_type=pl.DeviceIdType.MESH,
            )

        def compact_chunk(c, ord_ref, start, dst_ref):
            ords = ord_ref[pl.ds(c * CHUNK, CHUNK), :]
            col = lax.broadcasted_iota(jnp.int32, (CHUNK, W), 1) + start
            p = (ords == col).astype(jnp.bfloat16)
            rows = lax.dot_general(
                p, xb_ref[pl.ds(start, W), :], (((1,), (0,)), ((), ())),
                preferred_element_type=jnp.float32,
            )
            dst_ref[pl.ds(c * CHUNK, CHUNK), :] = rows.astype(jnp.bfloat16)

        for c in range(N_CHUNKS):
            @pl.when((c >= lo) & (c < hi))
            def _(c=c):
                start = pl.multiple_of(scal_ref[7 + c], 8)
                compact_chunk(c, sord_ref, start, buf_ref)
                chunk_rdma(c).start()

        for c in range(N_CHUNKS):
            @pl.when((c >= klo) & (c < khi))
            def _(c=c):
                start = pl.multiple_of(scal_ref[7 + N_CHUNKS + c], 8)
                compact_chunk(c, kord_ref, start, kbuf_ref)

        for c in range(N_CHUNKS):
            @pl.when((c >= rlo) & (c < rhi))
            def _(c=c):
                chunk_rdma(c).wait_recv()

        for c in range(N_CHUNKS):
            @pl.when((c >= lo) & (c < hi))
            def _(c=c):
                chunk_rdma(c).wait_send()

        keep_base = mx * ns
        row = lax.broadcasted_iota(jnp.int32, (ROWS, 1), 0)
        in_keep = (row >= keep_base) & (row < keep_base + nk)
        SLAB = 512
        for s in range(COLS // SLAB):
            cols = pl.ds(s * SLAB, SLAB)
            out_ref[:, cols] = jnp.where(
                in_keep, kbuf_ref[:, cols], recv_ref[:, cols]
            )

    return pl.pallas_call(
        body,
        out_shape=jax.ShapeDtypeStruct((ROWS, COLS), jnp.bfloat16),
        in_specs=[
            pl.BlockSpec(memory_space=pltpu.SMEM),
            pl.BlockSpec(memory_space=pltpu.VMEM),
            pl.BlockSpec(memory_space=pltpu.VMEM),
            pl.BlockSpec(memory_space=pltpu.VMEM),
        ],
        out_specs=pl.BlockSpec(memory_space=pltpu.VMEM),
        scratch_shapes=[
            pltpu.VMEM((ROWS, COLS), jnp.bfloat16),
            pltpu.VMEM((ROWS, COLS), jnp.bfloat16),
            pltpu.VMEM((ROWS, COLS), jnp.bfloat16),
            pltpu.SemaphoreType.DMA((N_CHUNKS,)),
            pltpu.SemaphoreType.DMA((N_CHUNKS,)),
        ],
        compiler_params=pltpu.CompilerParams(
            collective_id=0, vmem_limit_bytes=100 * 1024 * 1024
        ),
    )(scal, send_ord.reshape(ROWS, 1), keep_ord.reshape(ROWS, 1), xb)


# device time: 85141 ns/iter; 5.1646x vs baseline; 1.0902x over previous
import jax
import jax.numpy as jnp
from jax import lax
from jax.experimental import pallas as pl
from jax.experimental.pallas import tpu as pltpu

ROWS = 4096
COLS = 1024
CHUNK = 256
N_CHUNKS = ROWS // CHUNK
W = 768


def kernel(x, dest):
    my_x = lax.axis_index("x")

    keep = (dest == my_x).astype(jnp.int32)
    n_keep = jnp.sum(keep)
    ns = ROWS - n_keep
    order = jnp.argsort(keep, stable=True).astype(jnp.int32)
    ar = jnp.arange(ROWS, dtype=jnp.int32)

    rb = my_x * n_keep
    kb = my_x * ns
    spad = jnp.where(ar < ns, order, order[ns - 1])
    send_ord = jnp.where(ar < rb, order[0], jnp.roll(spad, rb))
    kpart = jnp.roll(order, -ns)
    kpad = jnp.where(ar < n_keep, kpart, order[ROWS - 1])
    keep_ord = jnp.where(ar < kb, order[ns], jnp.roll(kpad, kb))

    lo = rb // CHUNK
    hi = (rb + ns + CHUNK - 1) // CHUNK
    klo = kb // CHUNK
    khi = (kb + n_keep + CHUNK - 1) // CHUNK
    recv_base = (1 - my_x) * n_keep
    rlo = recv_base // CHUNK
    rhi = (recv_base + ns + CHUNK - 1) // CHUNK

    def starts_of(ord_arr):
        mins = ord_arr.reshape(N_CHUNKS, CHUNK).min(axis=1)
        return jnp.minimum((mins // 8) * 8, ROWS - W)

    scal = jnp.concatenate(
        [
            jnp.stack([ns, lo, hi, klo, khi, rlo, rhi]),
            starts_of(send_ord),
            starts_of(keep_ord),
        ]
    ).astype(jnp.int32)

    def body(scal_ref, sord_ref, kord_ref, x_ref, out_ref,
             xb_ref, buf_ref, recv_ref, send_sems, recv_sems):
        mx = lax.axis_index("x")
        my = lax.axis_index("y")
        mz = lax.axis_index("z")
        peer = (1 - mx, my, mz)

        ns = scal_ref[0]
        nk = ROWS - ns
        lo, hi = scal_ref[1], scal_ref[2]
        klo, khi = scal_ref[3], scal_ref[4]
        rlo, rhi = scal_ref[5], scal_ref[6]
        keep_base = mx * ns

        for s in range(COLS // 512):
            cols = pl.ds(s * 512, 512)
            xb_ref[:, cols] = x_ref[:, cols].astype(jnp.bfloat16)

        barrier_sem = pltpu.get_barrier_semaphore()
        pl.semaphore_signal(
            barrier_sem, inc=1, device_id=peer,
            device_id_type=pl.DeviceIdType.MESH,
        )
        pl.semaphore_wait(barrier_sem, 1)

        def chunk_rdma(i):
            return pltpu.make_async_remote_copy(
                src_ref=buf_ref.at[pl.ds(i * CHUNK, CHUNK), :],
                dst_ref=recv_ref.at[pl.ds(i * CHUNK, CHUNK), :],
                send_sem=send_sems.at[i],
                recv_sem=recv_sems.at[i],
                device_id=peer,
                device_id_type=pl.DeviceIdType.MESH,
            )

        def compact_chunk(c, ord_ref, start):
            ords = ord_ref[pl.ds(c * CHUNK, CHUNK), :]
            col = lax.broadcasted_iota(jnp.int32, (CHUNK, W), 1) + start
            p = (ords == col).astype(jnp.bfloat16)
            rows = lax.dot_general(
                p, xb_ref[pl.ds(start, W), :], (((1,), (0,)), ((), ())),
                preferred_element_type=jnp.float32,
            )
            return rows.astype(jnp.bfloat16)

        for c in range(N_CHUNKS):
            @pl.when((c >= lo) & (c < hi))
            def _(c=c):
                start = pl.multiple_of(scal_ref[7 + c], 8)
                buf_ref[pl.ds(c * CHUNK, CHUNK), :] = compact_chunk(
                    c, sord_ref, start
                )
                chunk_rdma(c).start()

        for c in range(N_CHUNKS):
            @pl.when((c >= klo) & (c < khi))
            def _(c=c):
                start = pl.multiple_of(scal_ref[7 + N_CHUNKS + c], 8)
                out_ref[pl.ds(c * CHUNK, CHUNK), :] = compact_chunk(
                    c, kord_ref, start
                )

        grow = lax.broadcasted_iota(jnp.int32, (CHUNK, 1), 0)
        for c in range(N_CHUNKS):
            @pl.when((c >= rlo) & (c < rhi))
            def _(c=c):
                chunk_rdma(c).wait_recv()
                rows = pl.ds(c * CHUNK, CHUNK)
                mask = ((grow + c * CHUNK) >= keep_base) & (
                    (grow + c * CHUNK) < keep_base + nk
                )
                out_ref[rows, :] = jnp.where(
                    mask, out_ref[rows, :], recv_ref[rows, :]
                )

        for c in range(N_CHUNKS):
            @pl.when((c >= lo) & (c < hi))
            def _(c=c):
                chunk_rdma(c).wait_send()

    return pl.pallas_call(
        body,
        out_shape=jax.ShapeDtypeStruct((ROWS, COLS), jnp.bfloat16),
        in_specs=[
            pl.BlockSpec(memory_space=pltpu.SMEM),
            pl.BlockSpec(memory_space=pltpu.VMEM),
            pl.BlockSpec(memory_space=pltpu.VMEM),
            pl.BlockSpec(memory_space=pltpu.VMEM),
        ],
        out_specs=pl.BlockSpec(memory_space=pltpu.VMEM),
        scratch_shapes=[
            pltpu.VMEM((ROWS, COLS), jnp.bfloat16),
            pltpu.VMEM((ROWS, COLS), jnp.bfloat16),
            pltpu.VMEM((ROWS, COLS), jnp.bfloat16),
            pltpu.SemaphoreType.DMA((N_CHUNKS,)),
            pltpu.SemaphoreType.DMA((N_CHUNKS,)),
        ],
        compiler_params=pltpu.CompilerParams(
            collective_id=0, vmem_limit_bytes=100 * 1024 * 1024
        ),
    )(scal, send_ord.reshape(ROWS, 1), keep_ord.reshape(ROWS, 1), x)
